# baseline (speedup 1.0000x reference)
import functools

import jax
import jax.numpy as jnp
from jax import lax
from jax.experimental import pallas as pl
from jax.experimental.pallas import tpu as pltpu
from jax.experimental.pallas import tpu_sc as plsc

N = 10000
E = 320000
D = 128
H = 128
C = 40
K = 3

NC = 2
NS = 16
NW = NC * NS
CHUNK = 128
NCHUNK = -(-E // (NW * CHUNK))
EPW = NCHUNK * CHUNK
E_PAD = EPW * NW
N_ACC = 10112
ROWS_PER_TILE = N_ACC // NS
FLUSH_TILES = 10
FLUSH_ROWS = N // FLUSH_TILES
BLK = 1000
NB = N // BLK

_mesh = plsc.VectorSubcoreMesh(core_axis_name="c", subcore_axis_name="s")
_f32 = jnp.float32


@functools.partial(
    pl.kernel,
    out_type=jax.ShapeDtypeStruct((NC * N, D), _f32),
    mesh=_mesh,
    scratch_types=[
        pltpu.VMEM_SHARED((N_ACC, D), _f32),
        pltpu.VMEM((CHUNK,), jnp.int32),
        pltpu.VMEM((CHUNK,), jnp.int32),
        pltpu.VMEM((CHUNK, D), _f32),
        pltpu.SemaphoreType.DMA,
    ],
)
def _sc_prop(z_hbm, srcp_hbm, dstp_hbm, zrows_hbm, out_hbm,
             acc, src_v, dst_v, rows_v, sem):
    c = lax.axis_index("c")
    s = lax.axis_index("s")
    wid = c * NS + s
    pltpu.sync_copy(zrows_hbm, acc.at[pl.ds(s * ROWS_PER_TILE, ROWS_PER_TILE)])
    plsc.subcore_barrier()
    base = wid * EPW

    def body(i, carry):
        off = base + i * CHUNK
        pltpu.sync_copy(srcp_hbm.at[pl.ds(off, CHUNK)], src_v)
        pltpu.sync_copy(dstp_hbm.at[pl.ds(off, CHUNK)], dst_v)
        pltpu.async_copy(z_hbm.at[src_v], rows_v, sem).wait()
        pltpu.sync_copy(rows_v, acc.at[dst_v], add=True)
        return carry

    lax.fori_loop(0, NCHUNK, body, 0)
    plsc.subcore_barrier()

    @pl.when(s < FLUSH_TILES)
    def _flush():
        r0 = s * FLUSH_ROWS
        pltpu.sync_copy(acc.at[pl.ds(r0, FLUSH_ROWS)],
                        out_hbm.at[pl.ds(c * N + r0, FLUSH_ROWS)])


def _row_spec(width):
    return pl.BlockSpec((BLK, width), lambda i: (i, 0))


def _half_specs(width):
    return [pl.BlockSpec((BLK, width), lambda i: (i, 0)),
            pl.BlockSpec((BLK, width), lambda i: (NB + i, 0))]


def _full_spec(r, c):
    return pl.BlockSpec((r, c), lambda i: (0, 0))


def _dot_t(a, w):
    return lax.dot_general(a, w, (((1,), (1,)), ((), ())),
                           preferred_element_type=_f32)


def _tc_dinv_body(d0_ref, d1_ref, dinv_ref, dinvsq_ref):
    deg = d0_ref[:, 0:1] + d1_ref[:, 0:1]
    pos = deg > 0
    dinv_ref[...] = jnp.where(pos, lax.rsqrt(deg), 0.0)
    dinvsq_ref[...] = jnp.where(pos, 1.0 / deg, 0.0)


def _tc_dinv(deg2):
    return pl.pallas_call(
        _tc_dinv_body,
        grid=(NB,),
        in_specs=_half_specs(D),
        out_specs=[_row_spec(1), _row_spec(1)],
        out_shape=[jax.ShapeDtypeStruct((N, 1), _f32),
                   jax.ShapeDtypeStruct((N, 1), _f32)],
    )(deg2, deg2)


def _tc_init_body(x_ref, dinv_ref, w_ref, b_ref, out_ref, z_ref):
    xb = x_ref[...]
    out_ref[...] = _dot_t(xb, w_ref[...]) + b_ref[...]
    z_ref[...] = dinv_ref[...] * xb


def _tc_init(x, dinv, w0, b):
    return pl.pallas_call(
        _tc_init_body,
        grid=(NB,),
        in_specs=[_row_spec(D), _row_spec(1), _full_spec(H, D),
                  _full_spec(1, H)],
        out_specs=[_row_spec(H), _row_spec(D)],
        out_shape=[jax.ShapeDtypeStruct((N, H), _f32),
                   jax.ShapeDtypeStruct((N, D), _f32)],
    )(x, dinv, w0, b.reshape(1, H))


def _tc_hop_mid_body(m0_ref, m1_ref, dinv_ref, dinvsq_ref, outin_ref, w_ref,
                     out_ref, z_ref):
    m = m0_ref[...] + m1_ref[...]
    out_ref[...] = outin_ref[...] + _dot_t(dinv_ref[...] * m, w_ref[...])
    z_ref[...] = dinvsq_ref[...] * m


def _tc_hop_mid(m2, dinv, dinvsq, outin, w):
    co = w.shape[0]
    return pl.pallas_call(
        _tc_hop_mid_body,
        grid=(NB,),
        in_specs=_half_specs(D) + [_row_spec(1), _row_spec(1), _row_spec(co),
                                   _full_spec(co, H)],
        out_specs=[_row_spec(co), _row_spec(D)],
        out_shape=[jax.ShapeDtypeStruct((N, co), _f32),
                   jax.ShapeDtypeStruct((N, D), _f32)],
    )(m2, m2, dinv, dinvsq, outin, w)


def _tc_hop_relu_body(m0_ref, m1_ref, dinv_ref, outin_ref, w_ref, wn_ref,
                      bn_ref, out_ref, z_ref):
    m = m0_ref[...] + m1_ref[...]
    dinv = dinv_ref[...]
    h = jnp.maximum(outin_ref[...] + _dot_t(dinv * m, w_ref[...]), 0.0)
    out_ref[...] = _dot_t(h, wn_ref[...]) + bn_ref[...]
    z_ref[...] = dinv * h


def _tc_hop_relu(m2, dinv, outin, w, w_next0, b_next):
    co = w.shape[0]
    cn = w_next0.shape[0]
    return pl.pallas_call(
        _tc_hop_relu_body,
        grid=(NB,),
        in_specs=_half_specs(D) + [_row_spec(1), _row_spec(co),
                                   _full_spec(co, H), _full_spec(cn, co),
                                   _full_spec(1, cn)],
        out_specs=[_row_spec(cn), _row_spec(D)],
        out_shape=[jax.ShapeDtypeStruct((N, cn), _f32),
                   jax.ShapeDtypeStruct((N, D), _f32)],
    )(m2, m2, dinv, outin, w, w_next0, b_next.reshape(1, cn))


def _tc_hop_final_body(m0_ref, m1_ref, dinv_ref, outin_ref, w_ref, out_ref):
    m = m0_ref[...] + m1_ref[...]
    o = outin_ref[...] + _dot_t(dinv_ref[...] * m, w_ref[...])
    mx = jnp.max(o, axis=1, keepdims=True)
    lse = mx + jnp.log(jnp.sum(jnp.exp(o - mx), axis=1, keepdims=True))
    out_ref[...] = o - lse


def _tc_hop_final(m2, dinv, outin, w):
    return pl.pallas_call(
        _tc_hop_final_body,
        grid=(NB,),
        in_specs=_half_specs(D) + [_row_spec(1), _row_spec(C),
                                   _full_spec(C, H)],
        out_specs=_row_spec(C),
        out_shape=jax.ShapeDtypeStruct((N, C), _f32),
    )(m2, m2, dinv, outin, w)


def kernel(x, edge_index, W1, b1, W2, b2, W3, b3):
    src = edge_index[0].astype(jnp.int32)
    dst = edge_index[1].astype(jnp.int32)
    order = jnp.argsort(src)
    src = src[order]
    dst = dst[order]
    pad = E_PAD - E
    pad_iota = jnp.arange(pad, dtype=jnp.int32)
    srcp = jnp.concatenate([src, pad_iota % N])
    dstp = jnp.concatenate([dst, N + pad_iota % (N_ACC - N)])
    zeros_d = jnp.zeros((ROWS_PER_TILE, D), _f32)
    ones_nd = jnp.ones((N, D), _f32)

    deg2 = _sc_prop(ones_nd, srcp, dstp, zeros_d)
    dinv, dinvsq = _tc_dinv(deg2)

    out, z = _tc_init(x, dinv, W1[0], b1)
    Ws = (W1, W2, W3)
    bs = (b1, b2, b3)
    for l in range(3):
        for k in range(1, K + 1):
            m2 = _sc_prop(z, srcp, dstp, zeros_d)
            if k < K:
                out, z = _tc_hop_mid(m2, dinv, dinvsq, out, Ws[l][k])
            elif l < 2:
                out, z = _tc_hop_relu(m2, dinv, out, Ws[l][k],
                                      Ws[l + 1][0], bs[l + 1])
            else:
                out = _tc_hop_final(m2, dinv, out, Ws[l][K])
    return out


# speedup vs baseline: 3.9574x; 3.9574x over previous
import functools

import jax
import jax.numpy as jnp
from jax import lax
from jax.experimental import pallas as pl
from jax.experimental.pallas import tpu as pltpu
from jax.experimental.pallas import tpu_sc as plsc

N = 10000
E = 320000
D = 128
H = 128
C = 40
K = 3

NC = 2
NS = 16
NW = NC * NS
CHUNK = 128
NCHUNK = 80
EPW = NCHUNK * CHUNK
E_PAD = EPW * NW
NBUF = 2
NROUND = NCHUNK // NBUF
N_ACC = 10112
ROWS_PER_TILE = N_ACC // NS
FLUSH_TILES = 10
FLUSH_ROWS = N // FLUSH_TILES
BLK = 1000
NB = N // BLK

_mesh = plsc.VectorSubcoreMesh(core_axis_name="c", subcore_axis_name="s")
_f32 = jnp.float32


@functools.partial(
    pl.kernel,
    out_type=jax.ShapeDtypeStruct((NC * N, D), _f32),
    mesh=_mesh,
    scratch_types=[
        pltpu.VMEM_SHARED((N_ACC, D), _f32),
        pltpu.VMEM((NCHUNK, CHUNK), jnp.int32),
        [pltpu.VMEM((CHUNK,), jnp.int32)] * NBUF,
        [pltpu.VMEM((CHUNK,), jnp.int32)] * NBUF,
        [pltpu.VMEM((CHUNK, D), _f32)] * NBUF,
        [pltpu.SemaphoreType.DMA] * NBUF,
        [pltpu.SemaphoreType.DMA] * NBUF,
    ],
)
def _sc_prop(z_hbm, packed_hbm, zrows_hbm, out_hbm,
             acc, packed_all, src_ring, dst_ring, bufs, gsems, ssems):
    c = lax.axis_index("c")
    s = lax.axis_index("s")
    wid = c * NS + s
    pltpu.sync_copy(zrows_hbm, acc.at[pl.ds(s * ROWS_PER_TILE, ROWS_PER_TILE)])
    pltpu.sync_copy(packed_hbm.at[pl.ds(wid * NCHUNK, NCHUNK)], packed_all)
    plsc.subcore_barrier()

    def unpack(i, b):
        for j in range(CHUNK // 16):
            p = packed_all[i, pl.ds(j * 16, 16)]
            src_ring[b][pl.ds(j * 16, 16)] = lax.bitwise_and(p, 0xFFFF)
            dst_ring[b][pl.ds(j * 16, 16)] = lax.shift_right_logical(p, 16)

    def gather_start(b):
        pltpu.async_copy(z_hbm.at[src_ring[b]], bufs[b], gsems[b])

    def gather_wait(b):
        pltpu.make_async_copy(z_hbm.at[src_ring[b]], bufs[b],
                              gsems[b]).wait()

    def scatter(b):
        return pltpu.async_copy(bufs[b], acc.at[dst_ring[b]], ssems[b],
                                add=True)

    for b in range(NBUF):
        unpack(b, b)
        gather_start(b)

    def round_body(g, carry):
        i0 = g * NBUF
        for b in range(NBUF):
            gather_wait(b)
            sdesc = scatter(b)
            sdesc.wait()
            unpack(i0 + NBUF + b, b)
            gather_start(b)
        return carry

    lax.fori_loop(0, NROUND - 1, round_body, 0)
    for b in range(NBUF):
        gather_wait(b)
        scatter(b).wait()
    plsc.subcore_barrier()

    @pl.when(s < FLUSH_TILES)
    def _flush():
        r0 = s * FLUSH_ROWS
        pltpu.sync_copy(acc.at[pl.ds(r0, FLUSH_ROWS)],
                        out_hbm.at[pl.ds(c * N + r0, FLUSH_ROWS)])


def _row_spec(width):
    return pl.BlockSpec((BLK, width), lambda i: (i, 0))


def _half_specs(width):
    return [pl.BlockSpec((BLK, width), lambda i: (i, 0)),
            pl.BlockSpec((BLK, width), lambda i: (NB + i, 0))]


def _full_spec(r, c):
    return pl.BlockSpec((r, c), lambda i: (0, 0))


def _dot_t(a, w):
    return lax.dot_general(a, w, (((1,), (1,)), ((), ())),
                           preferred_element_type=_f32)


def _tc_dinv_body(d0_ref, d1_ref, dinv_ref, dinvsq_ref):
    deg = d0_ref[:, 0:1] + d1_ref[:, 0:1]
    pos = deg > 0
    dinv_ref[...] = jnp.where(pos, lax.rsqrt(deg), 0.0)
    dinvsq_ref[...] = jnp.where(pos, 1.0 / deg, 0.0)


def _tc_dinv(deg2):
    return pl.pallas_call(
        _tc_dinv_body,
        grid=(NB,),
        in_specs=_half_specs(D),
        out_specs=[_row_spec(1), _row_spec(1)],
        out_shape=[jax.ShapeDtypeStruct((N, 1), _f32),
                   jax.ShapeDtypeStruct((N, 1), _f32)],
    )(deg2, deg2)


def _tc_init_body(x_ref, dinv_ref, w_ref, b_ref, out_ref, z_ref):
    xb = x_ref[...]
    out_ref[...] = _dot_t(xb, w_ref[...]) + b_ref[...]
    z_ref[...] = dinv_ref[...] * xb


def _tc_init(x, dinv, w0, b):
    return pl.pallas_call(
        _tc_init_body,
        grid=(NB,),
        in_specs=[_row_spec(D), _row_spec(1), _full_spec(H, D),
                  _full_spec(1, H)],
        out_specs=[_row_spec(H), _row_spec(D)],
        out_shape=[jax.ShapeDtypeStruct((N, H), _f32),
                   jax.ShapeDtypeStruct((N, D), _f32)],
    )(x, dinv, w0, b.reshape(1, H))


def _tc_hop_mid_body(m0_ref, m1_ref, dinv_ref, dinvsq_ref, outin_ref, w_ref,
                     out_ref, z_ref):
    m = m0_ref[...] + m1_ref[...]
    out_ref[...] = outin_ref[...] + _dot_t(dinv_ref[...] * m, w_ref[...])
    z_ref[...] = dinvsq_ref[...] * m


def _tc_hop_mid(m2, dinv, dinvsq, outin, w):
    co = w.shape[0]
    return pl.pallas_call(
        _tc_hop_mid_body,
        grid=(NB,),
        in_specs=_half_specs(D) + [_row_spec(1), _row_spec(1), _row_spec(co),
                                   _full_spec(co, H)],
        out_specs=[_row_spec(co), _row_spec(D)],
        out_shape=[jax.ShapeDtypeStruct((N, co), _f32),
                   jax.ShapeDtypeStruct((N, D), _f32)],
    )(m2, m2, dinv, dinvsq, outin, w)


def _tc_hop_relu_body(m0_ref, m1_ref, dinv_ref, outin_ref, w_ref, wn_ref,
                      bn_ref, out_ref, z_ref):
    m = m0_ref[...] + m1_ref[...]
    dinv = dinv_ref[...]
    h = jnp.maximum(outin_ref[...] + _dot_t(dinv * m, w_ref[...]), 0.0)
    out_ref[...] = _dot_t(h, wn_ref[...]) + bn_ref[...]
    z_ref[...] = dinv * h


def _tc_hop_relu(m2, dinv, outin, w, w_next0, b_next):
    co = w.shape[0]
    cn = w_next0.shape[0]
    return pl.pallas_call(
        _tc_hop_relu_body,
        grid=(NB,),
        in_specs=_half_specs(D) + [_row_spec(1), _row_spec(co),
                                   _full_spec(co, H), _full_spec(cn, co),
                                   _full_spec(1, cn)],
        out_specs=[_row_spec(cn), _row_spec(D)],
        out_shape=[jax.ShapeDtypeStruct((N, cn), _f32),
                   jax.ShapeDtypeStruct((N, D), _f32)],
    )(m2, m2, dinv, outin, w, w_next0, b_next.reshape(1, cn))


def _tc_hop_final_body(m0_ref, m1_ref, dinv_ref, outin_ref, w_ref, out_ref):
    m = m0_ref[...] + m1_ref[...]
    o = outin_ref[...] + _dot_t(dinv_ref[...] * m, w_ref[...])
    mx = jnp.max(o, axis=1, keepdims=True)
    lse = mx + jnp.log(jnp.sum(jnp.exp(o - mx), axis=1, keepdims=True))
    out_ref[...] = o - lse


def _tc_hop_final(m2, dinv, outin, w):
    return pl.pallas_call(
        _tc_hop_final_body,
        grid=(NB,),
        in_specs=_half_specs(D) + [_row_spec(1), _row_spec(C),
                                   _full_spec(C, H)],
        out_specs=_row_spec(C),
        out_shape=jax.ShapeDtypeStruct((N, C), _f32),
    )(m2, m2, dinv, outin, w)


def kernel(x, edge_index, W1, b1, W2, b2, W3, b3):
    src = edge_index[0].astype(jnp.int32)
    dst = edge_index[1].astype(jnp.int32)
    pad = E_PAD - E
    pad_iota = jnp.arange(pad, dtype=jnp.int32)
    srcp = jnp.concatenate([src, pad_iota % N])
    dstp = jnp.concatenate([dst, N + pad_iota % (N_ACC - N)])
    packed = jnp.bitwise_or(jnp.left_shift(dstp, 16), srcp
                            ).reshape(NW * NCHUNK, CHUNK)
    zeros_d = jnp.zeros((ROWS_PER_TILE, D), _f32)
    ones_nd = jnp.ones((N, D), _f32)

    deg2 = _sc_prop(ones_nd, packed, zeros_d)
    dinv, dinvsq = _tc_dinv(deg2)

    out, z = _tc_init(x, dinv, W1[0], b1)
    Ws = (W1, W2, W3)
    bs = (b1, b2, b3)
    for l in range(3):
        for k in range(1, K + 1):
            m2 = _sc_prop(z, packed, zeros_d)
            if k < K:
                out, z = _tc_hop_mid(m2, dinv, dinvsq, out, Ws[l][k])
            elif l < 2:
                out, z = _tc_hop_relu(m2, dinv, out, Ws[l][k],
                                      Ws[l + 1][0], bs[l + 1])
            else:
                out = _tc_hop_final(m2, dinv, out, Ws[l][K])
    return out
